# Initial kernel scaffold; baseline (speedup 1.0000x reference)
import jax
import jax.numpy as jnp
from jax.experimental import pallas as pl


def kernel(x, edge_index, partition, Wp, Wbb, Wbc, Wcb, Wcc, Wh):
    raise NotImplementedError("write your pallas kernel here")


# speedup vs baseline: 3.8023x; 3.8023x over previous
import functools

import jax
import jax.numpy as jnp
from jax import lax
from jax.experimental import pallas as pl
from jax.experimental.pallas import tpu as pltpu
from jax.experimental.pallas import tpu_sc as plsc

N = 10000
E = 320000
D = 128
C = 8
DW = 144
NPAD = 10240
NC, NS = 2, 16
NW = NC * NS
CHUNK = 128
EROWS = 2560
ROWS_PER_TILE = EROWS // NW
SLICE = NPAD // NS


def _prep_body(x_ref, wbb_ref, p_ref, xwa_ref, sumx_ref, counts_ref):
    x = x_ref[...]
    xw = jnp.dot(x, wbb_ref[...], preferred_element_type=jnp.float32)
    extra = jnp.where(
        lax.broadcasted_iota(jnp.int32, (N, DW - D), 1) == 0, 1.0, 0.0
    ).astype(jnp.float32)
    xwa_ref[...] = jnp.concatenate([xw, extra], axis=1)
    oh = (p_ref[...] == lax.broadcasted_iota(jnp.int32, (1, C), 1)).astype(jnp.float32)
    dn = (((0,), (0,)), ((), ()))
    sumx_ref[...] = lax.dot_general(oh, x, dn, preferred_element_type=jnp.float32)
    counts_ref[...] = lax.dot_general(
        oh, jnp.ones_like(x), dn, preferred_element_type=jnp.float32
    )


_prep = pl.pallas_call(
    _prep_body,
    out_shape=[
        jax.ShapeDtypeStruct((N, DW), jnp.float32),
        jax.ShapeDtypeStruct((C, D), jnp.float32),
        jax.ShapeDtypeStruct((C, D), jnp.float32),
    ],
)


def _sc_agg_body(xwa, src2d, dst2d, zeros, out, src_v, dst_v, buf, aggs, sem):
    cid = lax.axis_index("c")
    sid = lax.axis_index("s")
    wid = cid * NS + sid
    pltpu.sync_copy(zeros, aggs.at[pl.ds(sid * SLICE, SLICE)])
    pltpu.sync_copy(src2d.at[pl.ds(wid * ROWS_PER_TILE, ROWS_PER_TILE)], src_v)
    pltpu.sync_copy(dst2d.at[pl.ds(wid * ROWS_PER_TILE, ROWS_PER_TILE)], dst_v)
    plsc.subcore_barrier()

    @pl.loop(0, ROWS_PER_TILE)
    def _chunk(j):
        pltpu.async_copy(xwa.at[src_v.at[j]], buf, sem).wait()
        pltpu.sync_copy(buf, aggs.at[dst_v.at[j]], add=True)

    plsc.subcore_barrier()
    rows = pl.ds(sid * SLICE, SLICE)
    pltpu.sync_copy(aggs.at[rows], out.at[cid].at[rows])


@functools.cache
def _sc_agg():
    return pl.kernel(
        _sc_agg_body,
        out_type=jax.ShapeDtypeStruct((NC, NPAD, DW), jnp.float32),
        mesh=plsc.VectorSubcoreMesh(core_axis_name="c", subcore_axis_name="s"),
        scratch_types=[
            pltpu.VMEM((ROWS_PER_TILE, CHUNK), jnp.int32),
            pltpu.VMEM((ROWS_PER_TILE, CHUNK), jnp.int32),
            pltpu.VMEM((CHUNK, DW), jnp.float32),
            pltpu.VMEM_SHARED((NPAD, DW), jnp.float32),
            pltpu.SemaphoreType.DMA,
        ],
        compiler_params=pltpu.CompilerParams(use_tc_tiling_on_sc=False),
    )


def _epi_body(p0_ref, p1_ref, p_ref, sumx_ref, counts_ref,
              wp_ref, wbc_ref, wcb_ref, wcc_ref, wh_ref, out_ref):
    sumx = sumx_ref[...]
    cnt = jnp.maximum(counts_ref[...], 1.0)
    cmean = sumx / cnt
    centroid_x = jax.nn.relu(jnp.dot(cmean, wp_ref[...], preferred_element_type=jnp.float32))
    cwcb = jnp.dot(centroid_x, wcb_ref[...], preferred_element_type=jnp.float32)
    b2c = jnp.dot(cmean, wbc_ref[...], preferred_element_type=jnp.float32)
    cc = jnp.dot(
        (jnp.sum(centroid_x, axis=0, keepdims=True) - centroid_x) / (C - 1),
        wcc_ref[...], preferred_element_type=jnp.float32,
    )
    cent_emb = centroid_x + jax.nn.relu(b2c + cc)
    cent_mean = jnp.sum(cent_emb, axis=0, keepdims=True) / C

    a = p0_ref[:N, :] + p1_ref[:N, :]
    deg = jnp.maximum(a[:, D:D + 1], 1.0)
    bb = a[:, :D] / deg
    oh = (p_ref[...] == lax.broadcasted_iota(jnp.int32, (1, C), 1)).astype(jnp.float32)
    c2b = jnp.dot(oh, cwcb, preferred_element_type=jnp.float32)
    s = jax.nn.relu(bb + c2b)
    base_sum = jnp.sum(s, axis=0, keepdims=True)
    mean_x = jnp.sum(sumx, axis=0, keepdims=True) / N
    base_mean = mean_x + base_sum / N

    g = jnp.dot(
        jnp.concatenate([base_mean, cent_mean], axis=1),
        wh_ref[...], preferred_element_type=jnp.float32,
    )
    out_ref[...] = g


_epi = pl.pallas_call(
    _epi_body,
    out_shape=jax.ShapeDtypeStruct((1, D), jnp.float32),
)


def kernel(x, edge_index, partition, Wp, Wbb, Wbc, Wcb, Wcc, Wh):
    p2 = partition.reshape(N, 1)
    src = edge_index[0]
    dst = edge_index[1]
    src2d = jnp.concatenate(
        [src, jnp.zeros((EROWS * CHUNK - E,), jnp.int32)]).reshape(EROWS, CHUNK)
    dst2d = jnp.concatenate(
        [dst, jnp.full((EROWS * CHUNK - E,), N, jnp.int32)]).reshape(EROWS, CHUNK)
    zeros = jnp.zeros((SLICE, DW), jnp.float32)

    xwa, sumx, counts = _prep(x, Wbb, p2)
    part = _sc_agg()(xwa, src2d, dst2d, zeros)
    g = _epi(part[0], part[1], p2, sumx, counts, Wp, Wbc, Wcb, Wcc, Wh)
    return g.reshape(D)


# speedup vs baseline: 4.3486x; 1.1437x over previous
import functools

import jax
import jax.numpy as jnp
from jax import lax
from jax.experimental import pallas as pl
from jax.experimental.pallas import tpu as pltpu
from jax.experimental.pallas import tpu_sc as plsc

N = 10000
E = 320000
D = 128
C = 8
DW = 144
NPAD = 10016
NC, NS = 2, 16
NW = NC * NS
CHUNK = 64
EROWS = 5120
ROWS_PER_TILE = EROWS // NW
SLICE = NPAD // NS


def _prep_body(x_ref, wbb_ref, p_ref, xwa_ref, sumx_ref, counts_ref):
    x = x_ref[...]
    xw = jnp.dot(x, wbb_ref[...], preferred_element_type=jnp.float32)
    extra = jnp.where(
        lax.broadcasted_iota(jnp.int32, (N, DW - D), 1) == 0, 1.0, 0.0
    ).astype(jnp.float32)
    xwa_ref[...] = jnp.concatenate([xw, extra], axis=1)
    oh = (p_ref[...] == lax.broadcasted_iota(jnp.int32, (1, C), 1)).astype(jnp.float32)
    dn = (((0,), (0,)), ((), ()))
    sumx_ref[...] = lax.dot_general(oh, x, dn, preferred_element_type=jnp.float32)
    counts_ref[...] = lax.dot_general(
        oh, jnp.ones_like(x), dn, preferred_element_type=jnp.float32
    )


_prep = pl.pallas_call(
    _prep_body,
    out_shape=[
        jax.ShapeDtypeStruct((N, DW), jnp.float32),
        jax.ShapeDtypeStruct((C, D), jnp.float32),
        jax.ShapeDtypeStruct((C, D), jnp.float32),
    ],
)


def _sc_agg_body(xwa, src2d, dst2d, zeros, out, src_v, dst_v, buf0, buf1,
                 aggs, sem0, sem1):
    cid = lax.axis_index("c")
    sid = lax.axis_index("s")
    wid = cid * NS + sid
    pltpu.sync_copy(zeros, aggs.at[pl.ds(sid * SLICE, SLICE)])
    pltpu.sync_copy(src2d.at[pl.ds(wid * ROWS_PER_TILE, ROWS_PER_TILE)], src_v)
    pltpu.sync_copy(dst2d.at[pl.ds(wid * ROWS_PER_TILE, ROWS_PER_TILE)], dst_v)
    plsc.subcore_barrier()

    pltpu.async_copy(xwa.at[src_v.at[0]], buf0, sem0)

    @pl.loop(0, ROWS_PER_TILE, step=2)
    def _chunk(j):
        pltpu.async_copy(xwa.at[src_v.at[j + 1]], buf1, sem1)
        pltpu.make_async_copy(xwa.at[src_v.at[j]], buf0, sem0).wait()
        pltpu.sync_copy(buf0, aggs.at[dst_v.at[j]], add=True)

        @pl.when(j + 2 < ROWS_PER_TILE)
        def _():
            pltpu.async_copy(xwa.at[src_v.at[j + 2]], buf0, sem0)

        pltpu.make_async_copy(xwa.at[src_v.at[j + 1]], buf1, sem1).wait()
        pltpu.sync_copy(buf1, aggs.at[dst_v.at[j + 1]], add=True)

    plsc.subcore_barrier()
    rows = pl.ds(sid * SLICE, SLICE)
    pltpu.sync_copy(aggs.at[rows], out.at[cid].at[rows])


@functools.cache
def _sc_agg():
    return pl.kernel(
        _sc_agg_body,
        out_type=jax.ShapeDtypeStruct((NC, NPAD, DW), jnp.float32),
        mesh=plsc.VectorSubcoreMesh(core_axis_name="c", subcore_axis_name="s"),
        scratch_types=[
            pltpu.VMEM((ROWS_PER_TILE, CHUNK), jnp.int32),
            pltpu.VMEM((ROWS_PER_TILE, CHUNK), jnp.int32),
            pltpu.VMEM((CHUNK, DW), jnp.float32),
            pltpu.VMEM((CHUNK, DW), jnp.float32),
            pltpu.VMEM_SHARED((NPAD, DW), jnp.float32),
            pltpu.SemaphoreType.DMA,
            pltpu.SemaphoreType.DMA,
        ],
        compiler_params=pltpu.CompilerParams(use_tc_tiling_on_sc=False),
    )


def _epi_body(p0_ref, p1_ref, p_ref, sumx_ref, counts_ref,
              wp_ref, wbc_ref, wcb_ref, wcc_ref, wh_ref, out_ref):
    sumx = sumx_ref[...]
    cnt = jnp.maximum(counts_ref[...], 1.0)
    cmean = sumx / cnt
    centroid_x = jax.nn.relu(jnp.dot(cmean, wp_ref[...], preferred_element_type=jnp.float32))
    cwcb = jnp.dot(centroid_x, wcb_ref[...], preferred_element_type=jnp.float32)
    b2c = jnp.dot(cmean, wbc_ref[...], preferred_element_type=jnp.float32)
    cc = jnp.dot(
        (jnp.sum(centroid_x, axis=0, keepdims=True) - centroid_x) / (C - 1),
        wcc_ref[...], preferred_element_type=jnp.float32,
    )
    cent_emb = centroid_x + jax.nn.relu(b2c + cc)
    cent_mean = jnp.sum(cent_emb, axis=0, keepdims=True) / C

    a = p0_ref[:N, :] + p1_ref[:N, :]
    deg = jnp.maximum(a[:, D:D + 1], 1.0)
    bb = a[:, :D] / deg
    oh = (p_ref[...] == lax.broadcasted_iota(jnp.int32, (1, C), 1)).astype(jnp.float32)
    c2b = jnp.dot(oh, cwcb, preferred_element_type=jnp.float32)
    s = jax.nn.relu(bb + c2b)
    base_sum = jnp.sum(s, axis=0, keepdims=True)
    mean_x = jnp.sum(sumx, axis=0, keepdims=True) / N
    base_mean = mean_x + base_sum / N

    g = jnp.dot(
        jnp.concatenate([base_mean, cent_mean], axis=1),
        wh_ref[...], preferred_element_type=jnp.float32,
    )
    out_ref[...] = g


_epi = pl.pallas_call(
    _epi_body,
    out_shape=jax.ShapeDtypeStruct((1, D), jnp.float32),
)


def kernel(x, edge_index, partition, Wp, Wbb, Wbc, Wcb, Wcc, Wh):
    p2 = partition.reshape(N, 1)
    src = edge_index[0]
    dst = edge_index[1]
    src2d = jnp.concatenate(
        [src, jnp.zeros((EROWS * CHUNK - E,), jnp.int32)]).reshape(EROWS, CHUNK)
    dst2d = jnp.concatenate(
        [dst, jnp.full((EROWS * CHUNK - E,), N, jnp.int32)]).reshape(EROWS, CHUNK)
    zeros = jnp.zeros((SLICE, DW), jnp.float32)

    xwa, sumx, counts = _prep(x, Wbb, p2)
    part = _sc_agg()(xwa, src2d, dst2d, zeros)
    g = _epi(part[0], part[1], p2, sumx, counts, Wp, Wbc, Wcb, Wcc, Wh)
    return g.reshape(D)


# speedup vs baseline: 6.1558x; 1.4156x over previous
import functools

import numpy as np
import jax
import jax.numpy as jnp
from jax import lax
from jax.experimental import pallas as pl
from jax.experimental.pallas import tpu as pltpu
from jax.experimental.pallas import tpu_sc as plsc

N = 10000
E = 320000
D = 128
C = 8
DW = 160
NPAD = 10016
NC, NS = 2, 16
NW = NC * NS
CHUNK = 128
EROWS = 2560
RPT = EROWS // NW
SLICE = NPAD // NS
NBUF = 4


def _prep_body(x_ref, wbb_ref, p_ref, xwa_ref, sumx_ref, counts_ref):
    x = x_ref[...]
    xw = jnp.dot(x, wbb_ref[...], preferred_element_type=jnp.float32)
    extra = jnp.where(
        lax.broadcasted_iota(jnp.int32, (N, DW - D), 1) == 0, 1.0, 0.0
    ).astype(jnp.float32)
    xwa_ref[...] = jnp.concatenate([xw, extra], axis=1).astype(jnp.bfloat16)
    oh = (p_ref[...] == lax.broadcasted_iota(jnp.int32, (1, C), 1)).astype(jnp.float32)
    dn = (((0,), (0,)), ((), ()))
    sumx_ref[...] = lax.dot_general(oh, x, dn, preferred_element_type=jnp.float32)
    counts_ref[...] = lax.dot_general(
        oh, jnp.ones_like(x), dn, preferred_element_type=jnp.float32
    )


_prep = pl.pallas_call(
    _prep_body,
    out_shape=[
        jax.ShapeDtypeStruct((N, DW), jnp.bfloat16),
        jax.ShapeDtypeStruct((C, D), jnp.float32),
        jax.ShapeDtypeStruct((C, D), jnp.float32),
    ],
)


def _sc_agg_body(xwa, src2d, dst2d, zeros, out, src_v, dst_v,
                 b0, b1, b2, b3, aggs,
                 g0, g1, g2, g3, s0, s1, s2, s3):
    cid = lax.axis_index("c")
    sid = lax.axis_index("s")
    wid = cid * NS + sid
    bufs = [b0, b1, b2, b3]
    gsem = [g0, g1, g2, g3]
    ssem = [s0, s1, s2, s3]
    pltpu.sync_copy(zeros, aggs.at[pl.ds(sid * SLICE, SLICE)])
    pltpu.sync_copy(src2d.at[wid], src_v)
    pltpu.sync_copy(dst2d.at[wid], dst_v)
    plsc.subcore_barrier()

    def gather(c, k):
        pltpu.async_copy(xwa.at[src_v.at[c]], bufs[k], gsem[k])

    def wait_gather(c, k):
        pltpu.make_async_copy(xwa.at[src_v.at[c]], bufs[k], gsem[k]).wait()

    def scatter(c, k):
        pltpu.async_copy(bufs[k], aggs.at[dst_v.at[c]], ssem[k], add=True)

    def wait_scatter(k):
        pltpu.make_async_copy(bufs[k], aggs.at[dst_v.at[0]], ssem[k]).wait()

    for c in range(NBUF - 1):
        gather(c, c)
    wait_gather(0, 0)
    scatter(0, 0)
    gather(NBUF - 1, NBUF - 1)
    for k in range(1, NBUF):
        wait_gather(k, k)
        scatter(k, k)
        wait_scatter((k + NBUF - 1) % NBUF)
        gather(k + NBUF - 1, (k + NBUF - 1) % NBUF)

    @pl.loop(NBUF, RPT, step=NBUF)
    def _block(j):
        for k in range(NBUF):
            c = j + k
            wait_gather(c, k)
            scatter(c, k)

            @pl.when(c + NBUF - 1 < RPT)
            def _():
                wait_scatter((k + NBUF - 1) % NBUF)
                gather(c + NBUF - 1, (k + NBUF - 1) % NBUF)

    for k in range(NBUF):
        wait_scatter(k)

    plsc.subcore_barrier()
    rows = pl.ds(sid * SLICE, SLICE)
    pltpu.sync_copy(aggs.at[rows], out.at[cid].at[rows])


@functools.cache
def _sc_agg():
    return pl.kernel(
        _sc_agg_body,
        out_type=jax.ShapeDtypeStruct((NC, NPAD, DW), jnp.bfloat16),
        name="edge_agg_sc",
        mesh=plsc.VectorSubcoreMesh(core_axis_name="c", subcore_axis_name="s"),
        scratch_types=[
            pltpu.VMEM((RPT, CHUNK), jnp.int32),
            pltpu.VMEM((RPT, CHUNK), jnp.int32),
            pltpu.VMEM((CHUNK, DW), jnp.bfloat16),
            pltpu.VMEM((CHUNK, DW), jnp.bfloat16),
            pltpu.VMEM((CHUNK, DW), jnp.bfloat16),
            pltpu.VMEM((CHUNK, DW), jnp.bfloat16),
            pltpu.VMEM_SHARED((NPAD, DW), jnp.bfloat16),
            pltpu.SemaphoreType.DMA,
            pltpu.SemaphoreType.DMA,
            pltpu.SemaphoreType.DMA,
            pltpu.SemaphoreType.DMA,
            pltpu.SemaphoreType.DMA,
            pltpu.SemaphoreType.DMA,
            pltpu.SemaphoreType.DMA,
            pltpu.SemaphoreType.DMA,
        ],
        compiler_params=pltpu.CompilerParams(use_tc_tiling_on_sc=False),
    )


def _epi_body(part_ref, p_ref, sumx_ref, counts_ref,
              wp_ref, wbc_ref, wcb_ref, wcc_ref, wh_ref, out_ref):
    sumx = sumx_ref[...]
    cnt = jnp.maximum(counts_ref[...], 1.0)
    cmean = sumx / cnt
    centroid_x = jax.nn.relu(jnp.dot(cmean, wp_ref[...], preferred_element_type=jnp.float32))
    cwcb = jnp.dot(centroid_x, wcb_ref[...], preferred_element_type=jnp.float32)
    b2c = jnp.dot(cmean, wbc_ref[...], preferred_element_type=jnp.float32)
    cc = jnp.dot(
        (jnp.sum(centroid_x, axis=0, keepdims=True) - centroid_x) / (C - 1),
        wcc_ref[...], preferred_element_type=jnp.float32,
    )
    cent_emb = centroid_x + jax.nn.relu(b2c + cc)
    cent_mean = jnp.sum(cent_emb, axis=0, keepdims=True) / C

    a = (part_ref[0, :N, :].astype(jnp.float32)
         + part_ref[1, :N, :].astype(jnp.float32))
    deg = jnp.maximum(a[:, D:D + 1], 1.0)
    bb = a[:, :D] / deg
    oh = (p_ref[...] == lax.broadcasted_iota(jnp.int32, (1, C), 1)).astype(jnp.float32)
    c2b = jnp.dot(oh, cwcb, preferred_element_type=jnp.float32)
    s = jax.nn.relu(bb + c2b)
    base_sum = jnp.sum(s, axis=0, keepdims=True)
    mean_x = jnp.sum(sumx, axis=0, keepdims=True) / N
    base_mean = mean_x + base_sum / N

    g = jnp.dot(
        jnp.concatenate([base_mean, cent_mean], axis=1),
        wh_ref[...], preferred_element_type=jnp.float32,
    )
    out_ref[...] = g


_epi = pl.pallas_call(
    _epi_body,
    out_shape=jax.ShapeDtypeStruct((1, D), jnp.float32),
)


def kernel(x, edge_index, partition, Wp, Wbb, Wbc, Wcb, Wcc, Wh):
    p2 = partition.reshape(N, 1)
    src = edge_index[0]
    dst = edge_index[1]
    npad_e = EROWS * CHUNK - E
    sink = N + (jnp.arange(npad_e, dtype=jnp.int32) % (NPAD - N))
    src2d = jnp.concatenate(
        [src, jnp.zeros((npad_e,), jnp.int32)]
    ).reshape(RPT, NW, CHUNK).swapaxes(0, 1)
    dst2d = jnp.concatenate(
        [dst, sink]).reshape(RPT, NW, CHUNK).swapaxes(0, 1)
    zeros = jnp.zeros((SLICE, DW), jnp.bfloat16)

    xwa, sumx, counts = _prep(x, Wbb, p2)
    part = _sc_agg()(xwa, src2d, dst2d, zeros)
    g = _epi(part, p2, sumx, counts, Wp, Wbc, Wcb, Wcc, Wh)
    return g.reshape(D)


# speedup vs baseline: 6.2243x; 1.0111x over previous
import functools

import numpy as np
import jax
import jax.numpy as jnp
from jax import lax
from jax.experimental import pallas as pl
from jax.experimental.pallas import tpu as pltpu
from jax.experimental.pallas import tpu_sc as plsc

N = 10000
E = 320000
D = 128
C = 8
DW = 160
NPAD = 10016
NC, NS = 2, 16
NW = NC * NS
CHUNK = 128
EROWS = 2560
RPT = EROWS // NW
SLICE = NPAD // NS
NBUF = 4


def _prep_body(x_ref, wbb_ref, p_ref, xwa_ref, sumx_ref, counts_ref):
    x = x_ref[...]
    xw = jnp.dot(x, wbb_ref[...], preferred_element_type=jnp.float32)
    extra = jnp.where(
        lax.broadcasted_iota(jnp.int32, (N, DW - D), 1) == 0, 1.0, 0.0
    ).astype(jnp.float32)
    xwa_ref[...] = jnp.concatenate([xw, extra], axis=1).astype(jnp.bfloat16)
    oh = (p_ref[...] == lax.broadcasted_iota(jnp.int32, (1, C), 1)).astype(jnp.float32)
    dn = (((0,), (0,)), ((), ()))
    sumx_ref[...] = lax.dot_general(oh, x, dn, preferred_element_type=jnp.float32)
    counts_ref[...] = lax.dot_general(
        oh, jnp.ones_like(x), dn, preferred_element_type=jnp.float32
    )


_prep = pl.pallas_call(
    _prep_body,
    out_shape=[
        jax.ShapeDtypeStruct((N, DW), jnp.bfloat16),
        jax.ShapeDtypeStruct((C, D), jnp.float32),
        jax.ShapeDtypeStruct((C, D), jnp.float32),
    ],
)


def _sc_agg_body(xwa, src3d, dst3d, zeros, out, outd, src_v, dst_v,
                 b0, b1, b2, b3, aggs,
                 g0, g1, g2, g3, s0, s1, s2, s3):
    cid = lax.axis_index("c")
    sid = lax.axis_index("s")
    wid = cid * NS + sid
    bufs = [b0, b1, b2, b3]
    gsem = [g0, g1, g2, g3]
    ssem = [s0, s1, s2, s3]
    pltpu.sync_copy(zeros, aggs.at[pl.ds(sid * SLICE, SLICE)])
    pltpu.sync_copy(src3d.at[:, wid], src_v)
    pltpu.sync_copy(dst3d.at[:, wid], dst_v)
    plsc.subcore_barrier()

    def gather(c, k):
        pltpu.async_copy(xwa.at[src_v.at[c]], bufs[k], gsem[k])

    def wait_gather(c, k):
        pltpu.make_async_copy(xwa.at[src_v.at[c]], bufs[k], gsem[k]).wait()

    def scatter(c, k):
        pltpu.async_copy(bufs[k], aggs.at[dst_v.at[c]], ssem[k], add=True)

    def wait_scatter(k):
        pltpu.make_async_copy(bufs[k], aggs.at[dst_v.at[0]], ssem[k]).wait()

    for c in range(NBUF - 1):
        gather(c, c)
    wait_gather(0, 0)
    scatter(0, 0)
    gather(NBUF - 1, NBUF - 1)
    for k in range(1, NBUF):
        wait_gather(k, k)
        scatter(k, k)
        wait_scatter((k + NBUF - 1) % NBUF)
        gather(k + NBUF - 1, (k + NBUF - 1) % NBUF)

    @pl.loop(NBUF, RPT, step=NBUF)
    def _block(j):
        for k in range(NBUF):
            c = j + k
            wait_gather(c, k)
            scatter(c, k)

            @pl.when(c + NBUF - 1 < RPT)
            def _():
                wait_scatter((k + NBUF - 1) % NBUF)
                gather(c + NBUF - 1, (k + NBUF - 1) % NBUF)

    for k in range(NBUF):
        wait_scatter(k)

    plsc.subcore_barrier()
    rows = pl.ds(sid * SLICE, SLICE)
    pltpu.sync_copy(aggs.at[rows, pl.ds(0, D)], out.at[cid].at[rows])
    pltpu.sync_copy(aggs.at[rows, pl.ds(D, DW - D)], outd.at[cid].at[rows])


@functools.cache
def _sc_agg():
    return pl.kernel(
        _sc_agg_body,
        out_type=[
            jax.ShapeDtypeStruct((NC, NPAD, D), jnp.bfloat16),
            jax.ShapeDtypeStruct((NC, NPAD, DW - D), jnp.bfloat16),
        ],
        name="edge_agg_sc",
        mesh=plsc.VectorSubcoreMesh(core_axis_name="c", subcore_axis_name="s"),
        scratch_types=[
            pltpu.VMEM((RPT, CHUNK), jnp.int32),
            pltpu.VMEM((RPT, CHUNK), jnp.int32),
            pltpu.VMEM((CHUNK, DW), jnp.bfloat16),
            pltpu.VMEM((CHUNK, DW), jnp.bfloat16),
            pltpu.VMEM((CHUNK, DW), jnp.bfloat16),
            pltpu.VMEM((CHUNK, DW), jnp.bfloat16),
            pltpu.VMEM_SHARED((NPAD, DW), jnp.bfloat16),
            pltpu.SemaphoreType.DMA,
            pltpu.SemaphoreType.DMA,
            pltpu.SemaphoreType.DMA,
            pltpu.SemaphoreType.DMA,
            pltpu.SemaphoreType.DMA,
            pltpu.SemaphoreType.DMA,
            pltpu.SemaphoreType.DMA,
            pltpu.SemaphoreType.DMA,
        ],
        compiler_params=pltpu.CompilerParams(use_tc_tiling_on_sc=False),
    )


def _epi_body(part_ref, partd_ref, p_ref, sumx_ref, counts_ref,
              wp_ref, wbc_ref, wcb_ref, wcc_ref, wh_ref, out_ref):
    sumx = sumx_ref[...]
    cnt = jnp.maximum(counts_ref[...], 1.0)
    cmean = sumx / cnt
    centroid_x = jax.nn.relu(jnp.dot(cmean, wp_ref[...], preferred_element_type=jnp.float32))
    cwcb = jnp.dot(centroid_x, wcb_ref[...], preferred_element_type=jnp.float32)
    b2c = jnp.dot(cmean, wbc_ref[...], preferred_element_type=jnp.float32)
    cc = jnp.dot(
        (jnp.sum(centroid_x, axis=0, keepdims=True) - centroid_x) / (C - 1),
        wcc_ref[...], preferred_element_type=jnp.float32,
    )
    cent_emb = centroid_x + jax.nn.relu(b2c + cc)
    cent_mean = jnp.sum(cent_emb, axis=0, keepdims=True) / C

    a = (part_ref[0, :N, :].astype(jnp.float32)
         + part_ref[1, :N, :].astype(jnp.float32))
    ad = (partd_ref[0, :N, 0:1].astype(jnp.float32)
          + partd_ref[1, :N, 0:1].astype(jnp.float32))
    deg = jnp.maximum(ad, 1.0)
    bb = a / deg
    oh = (p_ref[...] == lax.broadcasted_iota(jnp.int32, (1, C), 1)).astype(jnp.float32)
    c2b = jnp.dot(oh, cwcb, preferred_element_type=jnp.float32)
    s = jax.nn.relu(bb + c2b)
    base_sum = jnp.sum(s, axis=0, keepdims=True)
    mean_x = jnp.sum(sumx, axis=0, keepdims=True) / N
    base_mean = mean_x + base_sum / N

    g = jnp.dot(
        jnp.concatenate([base_mean, cent_mean], axis=1),
        wh_ref[...], preferred_element_type=jnp.float32,
    )
    out_ref[...] = g


_epi = pl.pallas_call(
    _epi_body,
    out_shape=jax.ShapeDtypeStruct((1, D), jnp.float32),
)


def kernel(x, edge_index, partition, Wp, Wbb, Wbc, Wcb, Wcc, Wh):
    p2 = partition.reshape(N, 1)
    src = edge_index[0]
    dst = edge_index[1]
    npad_e = EROWS * CHUNK - E
    sink = N + (jnp.arange(npad_e, dtype=jnp.int32) % (NPAD - N))
    src3d = jnp.concatenate(
        [src, jnp.zeros((npad_e,), jnp.int32)]).reshape(RPT, NW, CHUNK)
    dst3d = jnp.concatenate([dst, sink]).reshape(RPT, NW, CHUNK)
    zeros = jnp.zeros((SLICE, DW), jnp.bfloat16)

    xwa, sumx, counts = _prep(x, Wbb, p2)
    part, partd = _sc_agg()(xwa, src3d, dst3d, zeros)
    g = _epi(part, partd, p2, sumx, counts, Wp, Wbc, Wcb, Wcc, Wh)
    return g.reshape(D)


# speedup vs baseline: 6.2686x; 1.0071x over previous
import functools

import numpy as np
import jax
import jax.numpy as jnp
from jax import lax
from jax.experimental import pallas as pl
from jax.experimental.pallas import tpu as pltpu
from jax.experimental.pallas import tpu_sc as plsc

N = 10000
E = 320000
D = 128
C = 8
DW = 160
NPAD = 10016
NC, NS = 2, 16
NW = NC * NS
CHUNK = 128
EROWS = 2560
RPT = EROWS // NW
SLICE = NPAD // NS
NBUF = 5


def _prep_body(x_ref, wbb_ref, p_ref, xwa_ref, sumx_ref, counts_ref):
    x = x_ref[...]
    xw = jnp.dot(x, wbb_ref[...], preferred_element_type=jnp.float32)
    extra = jnp.where(
        lax.broadcasted_iota(jnp.int32, (N, DW - D), 1) == 0, 1.0, 0.0
    ).astype(jnp.float32)
    xwa_ref[...] = jnp.concatenate([xw, extra], axis=1).astype(jnp.bfloat16)
    oh = (p_ref[...] == lax.broadcasted_iota(jnp.int32, (1, C), 1)).astype(jnp.float32)
    dn = (((0,), (0,)), ((), ()))
    sumx_ref[...] = lax.dot_general(oh, x, dn, preferred_element_type=jnp.float32)
    counts_ref[...] = lax.dot_general(
        oh, jnp.ones_like(x), dn, preferred_element_type=jnp.float32
    )


_prep = pl.pallas_call(
    _prep_body,
    out_shape=[
        jax.ShapeDtypeStruct((N, DW), jnp.bfloat16),
        jax.ShapeDtypeStruct((C, D), jnp.float32),
        jax.ShapeDtypeStruct((C, D), jnp.float32),
    ],
)


def _sc_agg_body(xwa, src3d, dst3d, zeros, out, outd, src_v, dst_v,
                 b0, b1, b2, b3, b4, aggs,
                 g0, g1, g2, g3, g4, s0, s1, s2, s3, s4):
    cid = lax.axis_index("c")
    sid = lax.axis_index("s")
    wid = cid * NS + sid
    bufs = [b0, b1, b2, b3, b4]
    gsem = [g0, g1, g2, g3, g4]
    ssem = [s0, s1, s2, s3, s4]
    pltpu.sync_copy(zeros, aggs.at[pl.ds(sid * SLICE, SLICE)])
    pltpu.sync_copy(src3d.at[:, wid], src_v)
    pltpu.sync_copy(dst3d.at[:, wid], dst_v)
    plsc.subcore_barrier()

    def gather(c, k):
        pltpu.async_copy(xwa.at[src_v.at[c]], bufs[k], gsem[k])

    def wait_gather(c, k):
        pltpu.make_async_copy(xwa.at[src_v.at[c]], bufs[k], gsem[k]).wait()

    def scatter(c, k):
        pltpu.async_copy(bufs[k], aggs.at[dst_v.at[c]], ssem[k], add=True)

    def wait_scatter(k):
        pltpu.make_async_copy(bufs[k], aggs.at[dst_v.at[0]], ssem[k]).wait()

    for c in range(NBUF - 1):
        gather(c, c)
    wait_gather(0, 0)
    scatter(0, 0)
    gather(NBUF - 1, NBUF - 1)
    for k in range(1, NBUF):
        wait_gather(k, k)
        scatter(k, k)
        wait_scatter((k + NBUF - 1) % NBUF)
        gather(k + NBUF - 1, (k + NBUF - 1) % NBUF)

    @pl.loop(NBUF, RPT, step=NBUF)
    def _block(j):
        for k in range(NBUF):
            c = j + k
            wait_gather(c, k)
            scatter(c, k)

            @pl.when(c + NBUF - 1 < RPT)
            def _():
                wait_scatter((k + NBUF - 1) % NBUF)
                gather(c + NBUF - 1, (k + NBUF - 1) % NBUF)

    for k in range(NBUF):
        wait_scatter(k)

    plsc.subcore_barrier()
    rows = pl.ds(sid * SLICE, SLICE)
    pltpu.sync_copy(aggs.at[rows, pl.ds(0, D)], out.at[cid].at[rows])
    pltpu.sync_copy(aggs.at[rows, pl.ds(D, DW - D)], outd.at[cid].at[rows])


@functools.cache
def _sc_agg():
    return pl.kernel(
        _sc_agg_body,
        out_type=[
            jax.ShapeDtypeStruct((NC, NPAD, D), jnp.bfloat16),
            jax.ShapeDtypeStruct((NC, NPAD, DW - D), jnp.bfloat16),
        ],
        name="edge_agg_sc",
        mesh=plsc.VectorSubcoreMesh(core_axis_name="c", subcore_axis_name="s"),
        scratch_types=[
            pltpu.VMEM((RPT, CHUNK), jnp.int32),
            pltpu.VMEM((RPT, CHUNK), jnp.int32),
            pltpu.VMEM((CHUNK, DW), jnp.bfloat16),
            pltpu.VMEM((CHUNK, DW), jnp.bfloat16),
            pltpu.VMEM((CHUNK, DW), jnp.bfloat16),
            pltpu.VMEM((CHUNK, DW), jnp.bfloat16),
            pltpu.VMEM((CHUNK, DW), jnp.bfloat16),
            pltpu.VMEM_SHARED((NPAD, DW), jnp.bfloat16),
        ] + [pltpu.SemaphoreType.DMA] * (2 * NBUF),
        compiler_params=pltpu.CompilerParams(use_tc_tiling_on_sc=False),
    )


def _epi_body(part_ref, partd_ref, p_ref, sumx_ref, counts_ref,
              wp_ref, wbc_ref, wcb_ref, wcc_ref, wh_ref, out_ref):
    sumx = sumx_ref[...]
    cnt = jnp.maximum(counts_ref[...], 1.0)
    cmean = sumx / cnt
    centroid_x = jax.nn.relu(jnp.dot(cmean, wp_ref[...], preferred_element_type=jnp.float32))
    cwcb = jnp.dot(centroid_x, wcb_ref[...], preferred_element_type=jnp.float32)
    b2c = jnp.dot(cmean, wbc_ref[...], preferred_element_type=jnp.float32)
    cc = jnp.dot(
        (jnp.sum(centroid_x, axis=0, keepdims=True) - centroid_x) / (C - 1),
        wcc_ref[...], preferred_element_type=jnp.float32,
    )
    cent_emb = centroid_x + jax.nn.relu(b2c + cc)
    cent_mean = jnp.sum(cent_emb, axis=0, keepdims=True) / C

    a = (part_ref[0, :N, :].astype(jnp.float32)
         + part_ref[1, :N, :].astype(jnp.float32))
    ad = (partd_ref[0, :N, 0:1].astype(jnp.float32)
          + partd_ref[1, :N, 0:1].astype(jnp.float32))
    deg = jnp.maximum(ad, 1.0)
    bb = a / deg
    oh = (p_ref[...] == lax.broadcasted_iota(jnp.int32, (1, C), 1)).astype(jnp.float32)
    c2b = jnp.dot(oh, cwcb, preferred_element_type=jnp.float32)
    s = jax.nn.relu(bb + c2b)
    base_sum = jnp.sum(s, axis=0, keepdims=True)
    mean_x = jnp.sum(sumx, axis=0, keepdims=True) / N
    base_mean = mean_x + base_sum / N

    g = jnp.dot(
        jnp.concatenate([base_mean, cent_mean], axis=1),
        wh_ref[...], preferred_element_type=jnp.float32,
    )
    out_ref[...] = g


_epi = pl.pallas_call(
    _epi_body,
    out_shape=jax.ShapeDtypeStruct((1, D), jnp.float32),
)


def kernel(x, edge_index, partition, Wp, Wbb, Wbc, Wcb, Wcc, Wh):
    p2 = partition.reshape(N, 1)
    src = edge_index[0]
    dst = edge_index[1]
    npad_e = EROWS * CHUNK - E
    sink = N + (jnp.arange(npad_e, dtype=jnp.int32) % (NPAD - N))
    src3d = jnp.concatenate(
        [src, jnp.zeros((npad_e,), jnp.int32)]).reshape(RPT, NW, CHUNK)
    dst3d = jnp.concatenate([dst, sink]).reshape(RPT, NW, CHUNK)
    zeros = jnp.zeros((SLICE, DW), jnp.bfloat16)

    xwa, sumx, counts = _prep(x, Wbb, p2)
    part, partd = _sc_agg()(xwa, src3d, dst3d, zeros)
    g = _epi(part, partd, p2, sumx, counts, Wp, Wbc, Wcb, Wcc, Wh)
    return g.reshape(D)


# speedup vs baseline: 7.3798x; 1.1773x over previous
import functools

import numpy as np
import jax
import jax.numpy as jnp
from jax import lax
from jax.experimental import pallas as pl
from jax.experimental.pallas import tpu as pltpu
from jax.experimental.pallas import tpu_sc as plsc

N = 10000
E = 320000
D = 128
C = 8
DEGW = 32
NPAD = 10016
NC, NS = 2, 16
NW = NC * NS
CHUNK = 128
EROWS = 2560
RPT = EROWS // NW
SLICE = NPAD // NS
NBUF = 5


def _prep_body(x_ref, wbb_ref, p_ref, xwa_ref, sumx_ref, counts_ref):
    x = x_ref[...]
    xw = jnp.dot(x, wbb_ref[...], preferred_element_type=jnp.float32)
    xwa_ref[...] = xw.astype(jnp.bfloat16)
    oh = (p_ref[...] == lax.broadcasted_iota(jnp.int32, (1, C), 1)).astype(jnp.float32)
    dn = (((0,), (0,)), ((), ()))
    sumx_ref[...] = lax.dot_general(oh, x, dn, preferred_element_type=jnp.float32)
    counts_ref[...] = lax.dot_general(
        oh, jnp.ones_like(x), dn, preferred_element_type=jnp.float32
    )


_prep = pl.pallas_call(
    _prep_body,
    out_shape=[
        jax.ShapeDtypeStruct((N, D), jnp.bfloat16),
        jax.ShapeDtypeStruct((C, D), jnp.float32),
        jax.ShapeDtypeStruct((C, D), jnp.float32),
    ],
)


def _sc_agg_body(xwa, src3d, dst3d, zf, zd, ones_h, out, outd, src_v, dst_v,
                 ones_v, b0, b1, b2, b3, b4, aggs, degs,
                 g0, g1, g2, g3, g4, s0, s1, s2, s3, s4,
                 d0, d1, d2, d3, d4):
    cid = lax.axis_index("c")
    sid = lax.axis_index("s")
    wid = cid * NS + sid
    bufs = [b0, b1, b2, b3, b4]
    gsem = [g0, g1, g2, g3, g4]
    ssem = [s0, s1, s2, s3, s4]
    dsem = [d0, d1, d2, d3, d4]
    pltpu.sync_copy(zf, aggs.at[pl.ds(sid * SLICE, SLICE)])
    pltpu.sync_copy(zd, degs.at[pl.ds(sid * SLICE, SLICE)])
    pltpu.sync_copy(ones_h, ones_v)
    pltpu.sync_copy(src3d.at[:, wid], src_v)
    pltpu.sync_copy(dst3d.at[:, wid], dst_v)
    plsc.subcore_barrier()

    def gather(c, k):
        pltpu.async_copy(xwa.at[src_v.at[c]], bufs[k], gsem[k])

    def wait_gather(c, k):
        pltpu.make_async_copy(xwa.at[src_v.at[c]], bufs[k], gsem[k]).wait()

    def scatter(c, k):
        pltpu.async_copy(bufs[k], aggs.at[dst_v.at[c]], ssem[k], add=True)
        pltpu.async_copy(ones_v, degs.at[dst_v.at[c]], dsem[k], add=True)

    def wait_scatter(k):
        pltpu.make_async_copy(bufs[k], aggs.at[dst_v.at[0]], ssem[k]).wait()
        pltpu.make_async_copy(ones_v, degs.at[dst_v.at[0]], dsem[k]).wait()

    for c in range(NBUF - 1):
        gather(c, c)
    wait_gather(0, 0)
    scatter(0, 0)
    gather(NBUF - 1, NBUF - 1)
    for k in range(1, NBUF):
        wait_gather(k, k)
        scatter(k, k)
        wait_scatter((k + NBUF - 1) % NBUF)
        gather(k + NBUF - 1, (k + NBUF - 1) % NBUF)

    @pl.loop(NBUF, RPT, step=NBUF)
    def _block(j):
        for k in range(NBUF):
            c = j + k
            wait_gather(c, k)
            scatter(c, k)

            @pl.when(c + NBUF - 1 < RPT)
            def _():
                wait_scatter((k + NBUF - 1) % NBUF)
                gather(c + NBUF - 1, (k + NBUF - 1) % NBUF)

    for k in range(NBUF):
        wait_scatter(k)

    plsc.subcore_barrier()
    rows = pl.ds(sid * SLICE, SLICE)
    pltpu.sync_copy(aggs.at[rows], out.at[cid].at[rows])
    pltpu.sync_copy(degs.at[rows], outd.at[cid].at[rows])


@functools.cache
def _sc_agg():
    return pl.kernel(
        _sc_agg_body,
        out_type=[
            jax.ShapeDtypeStruct((NC, NPAD, D), jnp.bfloat16),
            jax.ShapeDtypeStruct((NC, NPAD, DEGW), jnp.bfloat16),
        ],
        name="edge_agg_sc",
        mesh=plsc.VectorSubcoreMesh(core_axis_name="c", subcore_axis_name="s"),
        scratch_types=[
            pltpu.VMEM((RPT, CHUNK), jnp.int32),
            pltpu.VMEM((RPT, CHUNK), jnp.int32),
            pltpu.VMEM((CHUNK, DEGW), jnp.bfloat16),
            pltpu.VMEM((CHUNK, D), jnp.bfloat16),
            pltpu.VMEM((CHUNK, D), jnp.bfloat16),
            pltpu.VMEM((CHUNK, D), jnp.bfloat16),
            pltpu.VMEM((CHUNK, D), jnp.bfloat16),
            pltpu.VMEM((CHUNK, D), jnp.bfloat16),
            pltpu.VMEM_SHARED((NPAD, D), jnp.bfloat16),
            pltpu.VMEM_SHARED((NPAD, DEGW), jnp.bfloat16),
        ] + [pltpu.SemaphoreType.DMA] * (3 * NBUF),
        compiler_params=pltpu.CompilerParams(use_tc_tiling_on_sc=False),
    )


def _epi_body(part_ref, partd_ref, p_ref, sumx_ref, counts_ref,
              wp_ref, wbc_ref, wcb_ref, wcc_ref, wh_ref, out_ref):
    sumx = sumx_ref[...]
    cnt = jnp.maximum(counts_ref[...], 1.0)
    cmean = sumx / cnt
    centroid_x = jax.nn.relu(jnp.dot(cmean, wp_ref[...], preferred_element_type=jnp.float32))
    cwcb = jnp.dot(centroid_x, wcb_ref[...], preferred_element_type=jnp.float32)
    b2c = jnp.dot(cmean, wbc_ref[...], preferred_element_type=jnp.float32)
    cc = jnp.dot(
        (jnp.sum(centroid_x, axis=0, keepdims=True) - centroid_x) / (C - 1),
        wcc_ref[...], preferred_element_type=jnp.float32,
    )
    cent_emb = centroid_x + jax.nn.relu(b2c + cc)
    cent_mean = jnp.sum(cent_emb, axis=0, keepdims=True) / C

    a = (part_ref[0, :N, :].astype(jnp.float32)
         + part_ref[1, :N, :].astype(jnp.float32))
    ad = (partd_ref[0, :N, 0:1].astype(jnp.float32)
          + partd_ref[1, :N, 0:1].astype(jnp.float32))
    deg = jnp.maximum(ad, 1.0)
    bb = a / deg
    oh = (p_ref[...] == lax.broadcasted_iota(jnp.int32, (1, C), 1)).astype(jnp.float32)
    c2b = jnp.dot(oh, cwcb, preferred_element_type=jnp.float32)
    s = jax.nn.relu(bb + c2b)
    base_sum = jnp.sum(s, axis=0, keepdims=True)
    mean_x = jnp.sum(sumx, axis=0, keepdims=True) / N
    base_mean = mean_x + base_sum / N

    g = jnp.dot(
        jnp.concatenate([base_mean, cent_mean], axis=1),
        wh_ref[...], preferred_element_type=jnp.float32,
    )
    out_ref[...] = g


_epi = pl.pallas_call(
    _epi_body,
    out_shape=jax.ShapeDtypeStruct((1, D), jnp.float32),
)


def kernel(x, edge_index, partition, Wp, Wbb, Wbc, Wcb, Wcc, Wh):
    p2 = partition.reshape(N, 1)
    src = edge_index[0]
    dst = edge_index[1]
    npad_e = EROWS * CHUNK - E
    sink = N + (jnp.arange(npad_e, dtype=jnp.int32) % (NPAD - N))
    src3d = jnp.concatenate(
        [src, jnp.zeros((npad_e,), jnp.int32)]).reshape(RPT, NW, CHUNK)
    dst3d = jnp.concatenate([dst, sink]).reshape(RPT, NW, CHUNK)
    zf = jnp.zeros((SLICE, D), jnp.bfloat16)
    zd = jnp.zeros((SLICE, DEGW), jnp.bfloat16)
    ones_h = jnp.where(
        lax.broadcasted_iota(jnp.int32, (CHUNK, DEGW), 1) == 0, 1.0, 0.0
    ).astype(jnp.bfloat16)

    xwa, sumx, counts = _prep(x, Wbb, p2)
    part, partd = _sc_agg()(xwa, src3d, dst3d, zf, zd, ones_h)
    g = _epi(part, partd, p2, sumx, counts, Wp, Wbc, Wcb, Wcc, Wh)
    return g.reshape(D)
